# Initial kernel scaffold; baseline (speedup 1.0000x reference)
import jax
import jax.numpy as jnp
from jax.experimental import pallas as pl


def kernel(X):
    raise NotImplementedError("write your pallas kernel here")


# speedup vs baseline: 1.3217x; 1.3217x over previous
import functools

import jax
import jax.numpy as jnp
from jax import lax
from jax.experimental import pallas as pl
from jax.experimental.pallas import tpu as pltpu
from jax.experimental.pallas import tpu_sc as plsc

_CLASSES = 16
_COLS = 4
_FRAME = 38
_ALIGN = 8
_PADW = 48
_LANES = 16


def _block_origin(c, n):
    grid_num, frame_ratio = 512, 0.3
    frame_s = int(grid_num / _COLS * frame_ratio)
    g = grid_num
    row, col = c // _COLS, c % _COLS
    xc = int((n - g) // 2 + (col + 0.5) * (g // _COLS))
    yc = int((n - g) // 2 + (row + 0.5) * (g // _COLS))
    return yc - frame_s // 2, xc - frame_s // 2


def kernel(X):
    B, H, W = X.shape
    n = W
    origins = [_block_origin(c, n) for c in range(_CLASSES)]

    info = plsc.get_sparse_core_info()
    NC, NS = info.num_cores, info.num_subcores
    NW = NC * NS

    mesh = plsc.VectorSubcoreMesh(core_axis_name="c", subcore_axis_name="s")

    @functools.partial(
        pl.kernel,
        out_type=jax.ShapeDtypeStruct((B, _CLASSES), jnp.float32),
        mesh=mesh,
        compiler_params=pltpu.CompilerParams(
            use_tc_tiling_on_sc=False, needs_layout_passes=False),
        scratch_types=[
            pltpu.VMEM((_CLASSES, _FRAME, _PADW), jnp.float32),
            pltpu.VMEM((_LANES,), jnp.float32),
            pltpu.SemaphoreType.DMA,
        ],
    )
    def run(x_hbm, out_hbm, blocks_v, res_v, sem):
        wid = lax.axis_index("s") * NC + lax.axis_index("c")
        lane = lax.iota(jnp.int32, _LANES)
        skew = origins[0][1] % _ALIGN
        head_mask = lane >= skew
        tail_mask = lane < (skew + _FRAME - 2 * _LANES)

        for b0 in range(0, B, NW):
            b = b0 + wid
            copies = []
            for c in range(_CLASSES):
                y0, x0 = origins[c]
                copies.append(pltpu.make_async_copy(
                    x_hbm.at[b, pl.ds(y0, _FRAME), pl.ds(x0 - skew, _PADW)],
                    blocks_v.at[c],
                    sem,
                ))
            for cp in copies:
                cp.start()
            for cp in copies:
                cp.wait()

            res = jnp.zeros((_LANES,), jnp.float32)
            for c in range(_CLASSES):
                def body(r, acc):
                    v0 = blocks_v[c, r, pl.ds(0, _LANES)]
                    v1 = blocks_v[c, r, pl.ds(_LANES, _LANES)]
                    v2 = blocks_v[c, r, pl.ds(2 * _LANES, _LANES)]
                    v0 = jnp.where(head_mask, v0, 0.0)
                    v2 = jnp.where(tail_mask, v2, 0.0)
                    return acc + v0 * v0 + v1 * v1 + v2 * v2
                acc = lax.fori_loop(0, _FRAME, body, jnp.zeros((_LANES,), jnp.float32))
                s = jnp.sum(acc)
                res = jnp.where(lane == c, s, res)
            res_v[...] = res
            pltpu.sync_copy(res_v, out_hbm.at[b])

    return run(X)


# speedup vs baseline: 5.5033x; 4.1639x over previous
import functools

import jax
import jax.numpy as jnp
from jax import lax
from jax.experimental import pallas as pl
from jax.experimental.pallas import tpu as pltpu
from jax.experimental.pallas import tpu_sc as plsc

_CLASSES = 16
_COLS = 4
_FRAME = 38
_ROWT = 8
_COLT = 128
_ROWS = 48
_LANES = 16


def _block_origin(c, n):
    grid_num, frame_ratio = 512, 0.3
    frame_s = int(grid_num / _COLS * frame_ratio)
    g = grid_num
    row, col = c // _COLS, c % _COLS
    xc = int((n - g) // 2 + (col + 0.5) * (g // _COLS))
    yc = int((n - g) // 2 + (row + 0.5) * (g // _COLS))
    return yc - frame_s // 2, xc - frame_s // 2


def kernel(X):
    B, H, W = X.shape
    n = W
    origins = [_block_origin(c, n) for c in range(_CLASSES)]
    rskew = origins[0][0] % _ROWT
    cskew = origins[0][1] % _COLT
    assert all(y % _ROWT == rskew and x % _COLT == cskew for y, x in origins)
    assert rskew + _FRAME <= _ROWS

    info = plsc.get_sparse_core_info()
    NC, NS = info.num_cores, info.num_subcores
    NW = NC * NS

    mesh = plsc.VectorSubcoreMesh(core_axis_name="c", subcore_axis_name="s")

    @functools.partial(
        pl.kernel,
        out_type=jax.ShapeDtypeStruct((B, _CLASSES), jnp.float32),
        mesh=mesh,
        compiler_params=pltpu.CompilerParams(needs_layout_passes=False),
        scratch_types=[
            pltpu.VMEM((_CLASSES, _ROWS, _COLT), jnp.float32),
            pltpu.VMEM((_LANES,), jnp.float32),
            pltpu.SemaphoreType.DMA,
        ],
    )
    def run(x_hbm, out_hbm, blocks_v, res_v, sem):
        wid = lax.axis_index("s") * NC + lax.axis_index("c")
        lane = lax.iota(jnp.int32, _LANES)

        chunk0 = (cskew // _LANES) * _LANES
        nchunk = -(-(cskew + _FRAME - chunk0) // _LANES)
        masks = []
        for k in range(nchunk):
            lo, hi = chunk0 + k * _LANES, chunk0 + (k + 1) * _LANES
            if lo >= cskew and hi <= cskew + _FRAME:
                masks.append(None)
            else:
                masks.append((lane + lo >= cskew) & (lane + lo < cskew + _FRAME))

        for b0 in range(0, B, NW):
            b = b0 + wid
            copies = []
            for c in range(_CLASSES):
                y0, x0 = origins[c]
                copies.append(pltpu.make_async_copy(
                    x_hbm.at[b, pl.ds(y0 - rskew, _ROWS), pl.ds(x0 - cskew, _COLT)],
                    blocks_v.at[c],
                    sem,
                ))
            for cp in copies:
                cp.start()
            for cp in copies:
                cp.wait()

            res = jnp.zeros((_LANES,), jnp.float32)
            for c in range(_CLASSES):
                def body(r, acc):
                    for k in range(nchunk):
                        v = blocks_v[c, r, pl.ds(chunk0 + k * _LANES, _LANES)]
                        if masks[k] is not None:
                            v = jnp.where(masks[k], v, 0.0)
                        acc = acc + v * v
                    return acc
                acc = lax.fori_loop(rskew, rskew + _FRAME, body,
                                    jnp.zeros((_LANES,), jnp.float32))
                s = jnp.sum(acc)
                res = jnp.where(lane == c, s, res)
            res_v[...] = res
            pltpu.sync_copy(res_v, out_hbm.at[b])

    return run(X)


# speedup vs baseline: 5.7179x; 1.0390x over previous
import functools

import jax
import jax.numpy as jnp
from jax import lax
from jax.experimental import pallas as pl
from jax.experimental.pallas import tpu as pltpu
from jax.experimental.pallas import tpu_sc as plsc

_CLASSES = 16
_COLS = 4
_FRAME = 38
_ROWT = 8
_COLT = 128
_ROWS = 48
_LANES = 16


def _block_origin(c, n):
    grid_num, frame_ratio = 512, 0.3
    frame_s = int(grid_num / _COLS * frame_ratio)
    g = grid_num
    row, col = c // _COLS, c % _COLS
    xc = int((n - g) // 2 + (col + 0.5) * (g // _COLS))
    yc = int((n - g) // 2 + (row + 0.5) * (g // _COLS))
    return yc - frame_s // 2, xc - frame_s // 2


def kernel(X):
    B, H, W = X.shape
    n = W
    origins = [_block_origin(c, n) for c in range(_CLASSES)]
    rskew = origins[0][0] % _ROWT
    cskew = origins[0][1] % _COLT
    assert all(y % _ROWT == rskew and x % _COLT == cskew for y, x in origins)
    assert rskew + _FRAME <= _ROWS

    info = plsc.get_sparse_core_info()
    NC, NS = info.num_cores, info.num_subcores
    NW = NC * NS

    grid_x0 = origins[0][1] - cskew
    spanw = _COLS * _COLT
    row_starts = [origins[gr * _COLS][0] - rskew for gr in range(_COLS)]

    mesh = plsc.VectorSubcoreMesh(core_axis_name="c", subcore_axis_name="s")

    @functools.partial(
        pl.kernel,
        out_type=jax.ShapeDtypeStruct((B, _CLASSES), jnp.float32),
        mesh=mesh,
        compiler_params=pltpu.CompilerParams(needs_layout_passes=False),
        scratch_types=[
            pltpu.VMEM((_COLS, _ROWS, spanw), jnp.float32),
            pltpu.VMEM((_LANES,), jnp.float32),
            pltpu.SemaphoreType.DMA((_COLS,)),
        ],
    )
    def run(x_hbm, out_hbm, blocks_v, res_v, sems):
        wid = lax.axis_index("s") * NC + lax.axis_index("c")
        lane = lax.iota(jnp.int32, _LANES)

        tail_mask = lane < (_FRAME - 2 * _LANES)

        for b0 in range(0, B, NW):
            b = b0 + wid
            copies = []
            for gr in range(_COLS):
                copies.append(pltpu.make_async_copy(
                    x_hbm.at[b, pl.ds(row_starts[gr], _ROWS),
                             pl.ds(grid_x0, spanw)],
                    blocks_v.at[gr],
                    sems.at[gr],
                ))
            for cp in copies:
                cp.start()

            res = jnp.zeros((_LANES,), jnp.float32)
            for gr in range(_COLS):
                copies[gr].wait()
                for gc in range(_COLS):
                    cbase = gc * _COLT + cskew
                    def body(r, acc):
                        v0 = blocks_v[gr, r, pl.ds(cbase, _LANES)]
                        v1 = blocks_v[gr, r, pl.ds(cbase + _LANES, _LANES)]
                        v2 = blocks_v[gr, r, pl.ds(cbase + 2 * _LANES, _LANES)]
                        v2 = jnp.where(tail_mask, v2, 0.0)
                        return acc + v0 * v0 + v1 * v1 + v2 * v2
                    acc = lax.fori_loop(rskew, rskew + _FRAME, body,
                                        jnp.zeros((_LANES,), jnp.float32))
                    s = jnp.sum(acc)
                    res = jnp.where(lane == gr * _COLS + gc, s, res)
            res_v[...] = res
            pltpu.sync_copy(res_v, out_hbm.at[b])

    return run(X)


# speedup vs baseline: 5.7497x; 1.0056x over previous
import functools

import jax
import jax.numpy as jnp
from jax import lax
from jax.experimental import pallas as pl
from jax.experimental.pallas import tpu as pltpu
from jax.experimental.pallas import tpu_sc as plsc

_CLASSES = 16
_COLS = 4
_FRAME = 38
_ROWT = 8
_COLT = 128
_ROWS = 48
_LANES = 16


def _block_origin(c, n):
    grid_num, frame_ratio = 512, 0.3
    frame_s = int(grid_num / _COLS * frame_ratio)
    g = grid_num
    row, col = c // _COLS, c % _COLS
    xc = int((n - g) // 2 + (col + 0.5) * (g // _COLS))
    yc = int((n - g) // 2 + (row + 0.5) * (g // _COLS))
    return yc - frame_s // 2, xc - frame_s // 2


def kernel(X):
    B, H, W = X.shape
    n = W
    origins = [_block_origin(c, n) for c in range(_CLASSES)]
    rskew = origins[0][0] % _ROWT
    cskew = origins[0][1] % _COLT
    assert all(y % _ROWT == rskew and x % _COLT == cskew for y, x in origins)
    assert rskew + _FRAME <= _ROWS

    info = plsc.get_sparse_core_info()
    NC, NS = info.num_cores, info.num_subcores
    NW = NC * NS

    grid_x0 = origins[0][1] - cskew
    spanw = _COLS * _COLT
    row_starts = [origins[gr * _COLS][0] - rskew for gr in range(_COLS)]

    mesh = plsc.VectorSubcoreMesh(core_axis_name="c", subcore_axis_name="s")

    @functools.partial(
        pl.kernel,
        out_type=jax.ShapeDtypeStruct((B, _CLASSES), jnp.float32),
        mesh=mesh,
        compiler_params=pltpu.CompilerParams(needs_layout_passes=False),
        scratch_types=[
            pltpu.VMEM((_COLS, _ROWS, spanw), jnp.float32),
            pltpu.VMEM((_LANES,), jnp.float32),
            pltpu.SemaphoreType.DMA((_COLS,)),
        ],
    )
    def run(x_hbm, out_hbm, blocks_v, res_v, sems):
        wid = lax.axis_index("s") * NC + lax.axis_index("c")
        lane = lax.iota(jnp.int32, _LANES)

        tail_mask = lane < (_FRAME - 2 * _LANES)

        for b0 in range(0, B, NW):
            b = b0 + wid
            copies = []
            for gr in range(_COLS):
                copies.append(pltpu.make_async_copy(
                    x_hbm.at[b, pl.ds(row_starts[gr], _ROWS),
                             pl.ds(grid_x0, spanw)],
                    blocks_v.at[gr],
                    sems.at[gr],
                ))
            for cp in copies:
                cp.start()

            res = jnp.zeros((_LANES,), jnp.float32)
            zero = jnp.zeros((_LANES,), jnp.float32)
            for gr in range(_COLS):
                copies[gr].wait()
                for gc in range(_COLS):
                    cbase = gc * _COLT + cskew

                    def body(i, accs):
                        r = rskew + 2 * i
                        out = []
                        for dr in range(2):
                            for k in range(3):
                                v = blocks_v[gr, r + dr,
                                             pl.ds(cbase + k * _LANES, _LANES)]
                                out.append(accs[dr * 3 + k] + v * v)
                        return tuple(out)

                    accs = lax.fori_loop(0, _FRAME // 2, body, (zero,) * 6)
                    a0 = accs[0] + accs[3]
                    a1 = accs[1] + accs[4]
                    a2 = jnp.where(tail_mask, accs[2] + accs[5], 0.0)
                    s = jnp.sum(a0 + a1 + a2)
                    res = jnp.where(lane == gr * _COLS + gc, s, res)
            res_v[...] = res
            pltpu.sync_copy(res_v, out_hbm.at[b])

    return run(X)
